# Initial kernel scaffold; baseline (speedup 1.0000x reference)
import functools

import jax
import jax.numpy as jnp
from jax import lax
from jax.experimental import pallas as pl
from jax.experimental.pallas import tpu as pltpu
from jax.experimental.pallas import tpu_sc as plsc

N_USERS = 5000
N_ITEMS = 5000
N = N_USERS + N_ITEMS
E = 320000
D = 128
N_HOPS = 3

NC = 2
NS = 16
DH = D // NC
RPT = N // NS
RB = 125
NB = RPT // RB
CHUNK = 128
NCH = -(-E // (NS * CHUNK))
EPT = NCH * CHUNK

_mesh = plsc.VectorSubcoreMesh(core_axis_name="c", subcore_axis_name="s")


@functools.partial(
    pl.kernel,
    out_type=jax.ShapeDtypeStruct((NC, N, DH), jnp.float32),
    mesh=_mesh,
    scratch_types=[
        pltpu.VMEM((NCH, CHUNK), jnp.int32),
        pltpu.VMEM((NCH, CHUNK), jnp.int32),
        pltpu.VMEM((NCH, CHUNK), jnp.float32),
        pltpu.VMEM((CHUNK, DH), jnp.float32),
        pltpu.VMEM((RB, DH), jnp.float32),
        pltpu.VMEM((RB, DH), jnp.float32),
        pltpu.VMEM((RB, DH), jnp.float32),
        pltpu.VMEM((RB, DH), jnp.float32),
        pltpu.VMEM_SHARED((N, DH), jnp.float32),
        pltpu.VMEM_SHARED((N, DH), jnp.float32),
        pltpu.VMEM_SHARED((N, DH), jnp.float32),
    ],
)
def _sc_propagate(ego_hbm, col_hbm, row_hbm, val_hbm, out_hbm,
                  colbuf, rowbuf, valbuf, gbuf, zbuf, fb1, fb2, fb3,
                  cur, nxt, t3):
    c = lax.axis_index("c")
    s = lax.axis_index("s")
    row0 = s * RPT

    pltpu.sync_copy(col_hbm.at[s], colbuf)
    pltpu.sync_copy(row_hbm.at[s], rowbuf)
    pltpu.sync_copy(val_hbm.at[s], valbuf)
    pltpu.sync_copy(ego_hbm.at[c, pl.ds(row0, RPT)], cur.at[pl.ds(row0, RPT)])

    zeros16 = jnp.zeros((16,), jnp.float32)

    def zrow(i, carry):
        for q in range(DH // 16):
            zbuf[i, pl.ds(q * 16, 16)] = zeros16
        return carry

    lax.fori_loop(0, RB, zrow, 0)
    for b in range(NB):
        pltpu.sync_copy(zbuf, nxt.at[pl.ds(row0 + b * RB, RB)])
        pltpu.sync_copy(zbuf, t3.at[pl.ds(row0 + b * RB, RB)])
    plsc.subcore_barrier()

    lanes = jnp.arange(16, dtype=jnp.int32)

    def edge_pass(src, dst):
        def chunk(j, carry):
            pltpu.sync_copy(src.at[colbuf.at[j]], gbuf)

            def scale_g(g, carry2):
                v = valbuf[j, pl.ds(g * 16, 16)]
                ridx = lanes + g * 16
                for cidx in range(DH):
                    cvec = jnp.full((16,), cidx, jnp.int32)
                    x = plsc.load_gather(gbuf, [ridx, cvec])
                    plsc.store_scatter(gbuf, [ridx, cvec], x * v)
                return carry2

            lax.fori_loop(0, CHUNK // 16, scale_g, 0)
            pltpu.sync_copy(gbuf, dst.at[rowbuf.at[j]], add=True)
            return carry

        lax.fori_loop(0, NCH, chunk, 0)
        plsc.subcore_barrier()

    edge_pass(cur, nxt)
    for b in range(NB):
        pltpu.sync_copy(zbuf, cur.at[pl.ds(row0 + b * RB, RB)])
    plsc.subcore_barrier()
    edge_pass(nxt, cur)
    edge_pass(cur, t3)

    def crow(i, carry):
        for q in range(DH // 16):
            sl = pl.ds(q * 16, 16)
            fb1[i, sl] = (fb1[i, sl] + fb2[i, sl] + fb3[i, sl]) * (1.0 / 3.0)
        return carry

    for b in range(NB):
        r = row0 + b * RB
        pltpu.sync_copy(nxt.at[pl.ds(r, RB)], fb1)
        pltpu.sync_copy(cur.at[pl.ds(r, RB)], fb2)
        pltpu.sync_copy(t3.at[pl.ds(r, RB)], fb3)
        lax.fori_loop(0, RB, crow, 0)
        pltpu.sync_copy(fb1, out_hbm.at[c, pl.ds(r, RB)])


def kernel(user_embed, item_embed, adj_indices, adj_values):
    ego = jnp.concatenate([user_embed, item_embed], axis=0)
    ego_split = ego.reshape(N, NC, DH).transpose(1, 0, 2)
    row = adj_indices[0].astype(jnp.int32)
    col = adj_indices[1].astype(jnp.int32)
    val = adj_values.astype(jnp.float32)
    pad = NS * EPT - E
    colp = jnp.concatenate([col, jnp.zeros((pad,), jnp.int32)]).reshape(NS, NCH, CHUNK)
    rowp = jnp.concatenate([row, jnp.zeros((pad,), jnp.int32)]).reshape(NS, NCH, CHUNK)
    valp = jnp.concatenate([val, jnp.zeros((pad,), jnp.float32)]).reshape(NS, NCH, CHUNK)
    out = _sc_propagate(ego_split, colp, rowp, valp)
    mean_emb = out.transpose(1, 0, 2).reshape(N, D)
    return mean_emb[:N_USERS], mean_emb[N_USERS:]


# speedup vs baseline: 2.2512x; 2.2512x over previous
import functools

import jax
import jax.numpy as jnp
from jax import lax
from jax.experimental import pallas as pl
from jax.experimental.pallas import tpu as pltpu
from jax.experimental.pallas import tpu_sc as plsc

N_USERS = 5000
N_ITEMS = 5000
N = N_USERS + N_ITEMS
E = 320000
D = 128
N_HOPS = 3

NC = 2
NS = 16
DH = D // NC
NP = 10240
RPT = NP // NS
RB = 64
NB = RPT // RB
ZR = 32
CHUNK = 128
NCH = -(-E // (NS * CHUNK))
EPT = NCH * CHUNK

_mesh = plsc.VectorSubcoreMesh(core_axis_name="c", subcore_axis_name="s")


@functools.partial(
    pl.kernel,
    out_type=(
        jax.ShapeDtypeStruct((NC, NP, DH), jnp.float32),
        jax.ShapeDtypeStruct((NC, NP, DH), jnp.float32),
        jax.ShapeDtypeStruct((NC, NP, DH), jnp.float32),
    ),
    mesh=_mesh,
    compiler_params=pltpu.CompilerParams(use_tc_tiling_on_sc=False),
    scratch_types=[
        pltpu.VMEM((2, CHUNK), jnp.int32),
        pltpu.VMEM((CHUNK,), jnp.float32),
        pltpu.VMEM((CHUNK, DH), jnp.float32),
        pltpu.VMEM((ZR, DH), jnp.float32),
        pltpu.VMEM((RB, DH), jnp.float32),
        pltpu.VMEM((RB, DH), jnp.float32),
        pltpu.VMEM_SHARED((NP, DH), jnp.float32),
    ],
)
def _sc_propagate(ego_hbm, edges_hbm, vals_hbm, out_hbm, h1_hbm, h2_hbm,
                  ebuf, vbuf, gbuf, zbuf, fb1, fb2, acc):
    c = lax.axis_index("c")
    s = lax.axis_index("s")
    row0 = s * RPT

    zeros16 = jnp.zeros((16,), jnp.float32)

    def zrow(i, carry):
        for q in range(DH // 16):
            zbuf[i, pl.ds(q * 16, 16)] = zeros16
        return carry

    lax.fori_loop(0, ZR, zrow, 0)

    def zero_stripe():
        for z in range(RPT // ZR):
            pltpu.sync_copy(zbuf, acc.at[pl.ds(row0 + z * ZR, ZR)])

    zero_stripe()
    plsc.subcore_barrier()

    def edge_pass(src):
        def chunk(j, carry):
            pltpu.sync_copy(edges_hbm.at[s, j], ebuf)
            pltpu.sync_copy(vals_hbm.at[s, j], vbuf)
            pltpu.sync_copy(src.at[ebuf.at[0]], gbuf)

            def scale_g(g, carry2):
                v16 = vbuf[pl.ds(g * 16, 16)]
                for k in range(16):
                    e = g * 16 + k
                    v = jnp.full((16,), v16[k])
                    for q in range(DH // 16):
                        sl = pl.ds(q * 16, 16)
                        gbuf[e, sl] = gbuf[e, sl] * v
                return carry2

            lax.fori_loop(0, CHUNK // 16, scale_g, 0)
            pltpu.sync_copy(gbuf, acc.at[ebuf.at[1]], add=True)
            return carry

        lax.fori_loop(0, NCH, chunk, 0)
        plsc.subcore_barrier()

    def dump_and_clear(dst_hbm):
        for b in range(NB):
            r = row0 + b * RB
            pltpu.sync_copy(acc.at[pl.ds(r, RB)], fb1)
            pltpu.sync_copy(fb1, dst_hbm.at[c, pl.ds(r, RB)])
        zero_stripe()
        plsc.subcore_barrier()

    edge_pass(ego_hbm.at[c])
    dump_and_clear(h1_hbm)
    edge_pass(h1_hbm.at[c])
    dump_and_clear(h2_hbm)
    edge_pass(h2_hbm.at[c])

    gb = gbuf.at[pl.ds(0, RB)]

    def crow(i, carry):
        for q in range(DH // 16):
            sl = pl.ds(q * 16, 16)
            fb1[i, sl] = (fb1[i, sl] + fb2[i, sl] + gbuf[i, sl]) * (1.0 / 3.0)
        return carry

    for b in range(NB):
        r = row0 + b * RB
        pltpu.sync_copy(h1_hbm.at[c, pl.ds(r, RB)], fb1)
        pltpu.sync_copy(h2_hbm.at[c, pl.ds(r, RB)], fb2)
        pltpu.sync_copy(acc.at[pl.ds(r, RB)], gb)
        lax.fori_loop(0, RB, crow, 0)
        pltpu.sync_copy(fb1, out_hbm.at[c, pl.ds(r, RB)])


def kernel(user_embed, item_embed, adj_indices, adj_values):
    ego = jnp.concatenate([user_embed, item_embed], axis=0)
    ego_split = ego.reshape(N, NC, DH).transpose(1, 0, 2)
    ego_pad = jnp.concatenate(
        [ego_split, jnp.zeros((NC, NP - N, DH), jnp.float32)], axis=1
    )
    row = adj_indices[0].astype(jnp.int32)
    col = adj_indices[1].astype(jnp.int32)
    val = adj_values.astype(jnp.float32)
    pad = NS * EPT - E
    colp = jnp.concatenate([col, jnp.zeros((pad,), jnp.int32)]).reshape(NS, NCH, CHUNK)
    rowp = jnp.concatenate([row, jnp.zeros((pad,), jnp.int32)]).reshape(NS, NCH, CHUNK)
    valp = jnp.concatenate([val, jnp.zeros((pad,), jnp.float32)]).reshape(NS, NCH, CHUNK)
    edges = jnp.stack([colp, rowp], axis=2)
    out, _h1, _h2 = _sc_propagate(ego_pad, edges, valp)
    mean_emb = out[:, :N, :].transpose(1, 0, 2).reshape(N, D)
    return mean_emb[:N_USERS], mean_emb[N_USERS:]


# speedup vs baseline: 2.7844x; 1.2368x over previous
import functools

import jax
import jax.numpy as jnp
from jax import lax
from jax.experimental import pallas as pl
from jax.experimental.pallas import tpu as pltpu
from jax.experimental.pallas import tpu_sc as plsc

N_USERS = 5000
N_ITEMS = 5000
N = N_USERS + N_ITEMS
E = 320000
D = 128
N_HOPS = 3

NC = 2
NS = 16
DH = D // NC
NP = 10240
RPT = NP // NS
RB = 32
NB = RPT // RB
CHUNK = 128
NCH = 158
EPT = NCH * CHUNK

_mesh = plsc.VectorSubcoreMesh(core_axis_name="c", subcore_axis_name="s")


@functools.partial(
    pl.kernel,
    out_type=(
        jax.ShapeDtypeStruct((NC, NP, DH), jnp.float32),
        jax.ShapeDtypeStruct((NC, NP, DH), jnp.float32),
        jax.ShapeDtypeStruct((NC, NP, DH), jnp.float32),
    ),
    mesh=_mesh,
    compiler_params=pltpu.CompilerParams(use_tc_tiling_on_sc=False),
    scratch_types=[
        pltpu.VMEM((2, CHUNK), jnp.int32),
        pltpu.VMEM((CHUNK,), jnp.float32),
        pltpu.VMEM((CHUNK, DH), jnp.float32),
        pltpu.VMEM((2, CHUNK), jnp.int32),
        pltpu.VMEM((CHUNK,), jnp.float32),
        pltpu.VMEM((CHUNK, DH), jnp.float32),
        pltpu.VMEM((RB, DH), jnp.float32),
        pltpu.SemaphoreType.DMA,
        pltpu.SemaphoreType.DMA,
        pltpu.VMEM_SHARED((NP, DH), jnp.float32),
    ],
)
def _sc_propagate(ego_hbm, edges_hbm, vals_hbm, out_hbm, h1_hbm, h2_hbm,
                  ebuf0, vbuf0, gbuf0, ebuf1, vbuf1, gbuf1, zbuf,
                  sem0, sem1, acc):
    c = lax.axis_index("c")
    s = lax.axis_index("s")
    row0 = s * RPT

    zeros16 = jnp.zeros((16,), jnp.float32)

    def zrow(i, carry):
        for q in range(DH // 16):
            zbuf[i, pl.ds(q * 16, 16)] = zeros16
        return carry

    lax.fori_loop(0, RB, zrow, 0)

    def zero_stripe():
        for z in range(NB):
            pltpu.sync_copy(zbuf, acc.at[pl.ds(row0 + z * RB, RB)])

    zero_stripe()
    plsc.subcore_barrier()

    def scale(gbuf, vbuf):
        def scale_g(g, carry):
            v16 = vbuf[pl.ds(g * 16, 16)]
            for k in range(16):
                e = g * 16 + k
                v = jnp.full((16,), v16[k])
                for q in range(DH // 16):
                    sl = pl.ds(q * 16, 16)
                    gbuf[e, sl] = gbuf[e, sl] * v
            return carry

        lax.fori_loop(0, CHUNK // 16, scale_g, 0)

    def edge_pass(src):
        pltpu.sync_copy(edges_hbm.at[s, 0], ebuf0)
        pltpu.sync_copy(vals_hbm.at[s, 0], vbuf0)
        pltpu.async_copy(src.at[ebuf0.at[0]], gbuf0, sem0)

        def pair(p, carry):
            jB = 2 * p + 1
            jN = jnp.minimum(jB + 1, NCH - 1)
            pltpu.sync_copy(edges_hbm.at[s, jB], ebuf1)
            pltpu.sync_copy(vals_hbm.at[s, jB], vbuf1)
            pltpu.async_copy(src.at[ebuf1.at[0]], gbuf1, sem1)
            pltpu.make_async_copy(src.at[ebuf0.at[0]], gbuf0, sem0).wait()
            scale(gbuf0, vbuf0)
            pltpu.sync_copy(gbuf0, acc.at[ebuf0.at[1]], add=True)
            pltpu.sync_copy(edges_hbm.at[s, jN], ebuf0)
            pltpu.sync_copy(vals_hbm.at[s, jN], vbuf0)
            pltpu.async_copy(src.at[ebuf0.at[0]], gbuf0, sem0)
            pltpu.make_async_copy(src.at[ebuf1.at[0]], gbuf1, sem1).wait()
            scale(gbuf1, vbuf1)
            pltpu.sync_copy(gbuf1, acc.at[ebuf1.at[1]], add=True)
            return carry

        lax.fori_loop(0, NCH // 2, pair, 0)
        pltpu.make_async_copy(src.at[ebuf0.at[0]], gbuf0, sem0).wait()
        plsc.subcore_barrier()

    gb0 = gbuf0.at[pl.ds(0, RB)]
    gb1 = gbuf1.at[pl.ds(0, RB)]

    def dump_and_clear(dst_hbm):
        for b in range(NB):
            r = row0 + b * RB
            pltpu.sync_copy(acc.at[pl.ds(r, RB)], gb0)
            pltpu.sync_copy(gb0, dst_hbm.at[c, pl.ds(r, RB)])
        zero_stripe()
        plsc.subcore_barrier()

    edge_pass(ego_hbm.at[c])
    dump_and_clear(h1_hbm)
    edge_pass(h1_hbm.at[c])
    dump_and_clear(h2_hbm)
    edge_pass(h2_hbm.at[c])

    def crow(i, carry):
        for q in range(DH // 16):
            sl = pl.ds(q * 16, 16)
            gbuf0[i, sl] = (gbuf0[i, sl] + gbuf1[i, sl] + zbuf[i, sl]) * (1.0 / 3.0)
        return carry

    for b in range(NB):
        r = row0 + b * RB
        pltpu.sync_copy(h1_hbm.at[c, pl.ds(r, RB)], gb0)
        pltpu.sync_copy(h2_hbm.at[c, pl.ds(r, RB)], gb1)
        pltpu.sync_copy(acc.at[pl.ds(r, RB)], zbuf)
        lax.fori_loop(0, RB, crow, 0)
        pltpu.sync_copy(gb0, out_hbm.at[c, pl.ds(r, RB)])


def kernel(user_embed, item_embed, adj_indices, adj_values):
    ego = jnp.concatenate([user_embed, item_embed], axis=0)
    ego_split = ego.reshape(N, NC, DH).transpose(1, 0, 2)
    ego_pad = jnp.concatenate(
        [ego_split, jnp.zeros((NC, NP - N, DH), jnp.float32)], axis=1
    )
    row = adj_indices[0].astype(jnp.int32)
    col = adj_indices[1].astype(jnp.int32)
    val = adj_values.astype(jnp.float32)
    pad = NS * EPT - E
    colp = jnp.concatenate([col, jnp.zeros((pad,), jnp.int32)]).reshape(NS, NCH, CHUNK)
    rowp = jnp.concatenate([row, jnp.zeros((pad,), jnp.int32)]).reshape(NS, NCH, CHUNK)
    valp = jnp.concatenate([val, jnp.zeros((pad,), jnp.float32)]).reshape(NS, NCH, CHUNK)
    edges = jnp.stack([colp, rowp], axis=2)
    out, _h1, _h2 = _sc_propagate(ego_pad, edges, valp)
    mean_emb = out[:, :N, :].transpose(1, 0, 2).reshape(N, D)
    return mean_emb[:N_USERS], mean_emb[N_USERS:]


# speedup vs baseline: 3.7493x; 1.3466x over previous
import functools

import jax
import jax.numpy as jnp
from jax import lax
from jax.experimental import pallas as pl
from jax.experimental.pallas import tpu as pltpu
from jax.experimental.pallas import tpu_sc as plsc

N_USERS = 5000
N_ITEMS = 5000
N = N_USERS + N_ITEMS
E = 320000
D = 128
N_HOPS = 3

NC = 2
NS = 16
DH = D // NC
NP = 10240
RPT = NP // NS
RB = 32
NB = RPT // RB
CHUNK = 128
NCH = 160
NPAIR = NCH // 2
EPT = NCH * CHUNK

_mesh = plsc.VectorSubcoreMesh(core_axis_name="c", subcore_axis_name="s")


@functools.partial(
    pl.kernel,
    out_type=(
        jax.ShapeDtypeStruct((NC, NP, DH), jnp.float32),
        jax.ShapeDtypeStruct((NC, NP, DH), jnp.float32),
        jax.ShapeDtypeStruct((NC, NP, DH), jnp.float32),
    ),
    mesh=_mesh,
    compiler_params=pltpu.CompilerParams(use_tc_tiling_on_sc=False),
    scratch_types=[
        pltpu.VMEM((2, 2, CHUNK), jnp.int32),
        pltpu.VMEM((2, CHUNK), jnp.float32),
        pltpu.VMEM((CHUNK, DH), jnp.float32),
        pltpu.VMEM((2, 2, CHUNK), jnp.int32),
        pltpu.VMEM((2, CHUNK), jnp.float32),
        pltpu.VMEM((CHUNK, DH), jnp.float32),
        pltpu.VMEM((RB, DH), jnp.float32),
        pltpu.SemaphoreType.DMA,
        pltpu.SemaphoreType.DMA,
        pltpu.SemaphoreType.DMA,
        pltpu.SemaphoreType.DMA,
        pltpu.VMEM_SHARED((NP, DH), jnp.float32),
    ],
)
def _sc_propagate(ego_hbm, edges_hbm, vals_hbm, out_hbm, h1_hbm, h2_hbm,
                  ebuf0, vbuf0, gbuf0, ebuf1, vbuf1, gbuf1, zbuf,
                  sem0, sem1, esem0, esem1, acc):
    c = lax.axis_index("c")
    s = lax.axis_index("s")
    row0 = s * RPT

    zeros16 = jnp.zeros((16,), jnp.float32)

    def zrow(i, carry):
        for q in range(DH // 16):
            zbuf[i, pl.ds(q * 16, 16)] = zeros16
        return carry

    lax.fori_loop(0, RB, zrow, 0)

    def zero_stripe():
        for z in range(NB):
            pltpu.sync_copy(zbuf, acc.at[pl.ds(row0 + z * RB, RB)])

    zero_stripe()
    plsc.subcore_barrier()

    def scale(gbuf, vbuf):
        def scale_g(g, carry):
            v16 = vbuf[pl.ds(g * 16, 16)]
            for k in range(16):
                e = g * 16 + k
                v = jnp.full((16,), v16[k])
                for q in range(DH // 16):
                    sl = pl.ds(q * 16, 16)
                    gbuf[e, sl] = gbuf[e, sl] * v
            return carry

        lax.fori_loop(0, CHUNK // 16, scale_g, 0)

    def edge_pass(src):
        pltpu.sync_copy(edges_hbm.at[s, 0], ebuf0)
        pltpu.sync_copy(vals_hbm.at[s, 0], vbuf0)
        pltpu.async_copy(src.at[ebuf0.at[0, 0]], gbuf0, sem0)

        def quad(q, carry):
            pn = jnp.minimum(2 * q + 2, NPAIR - 1)
            pltpu.async_copy(edges_hbm.at[s, 2 * q + 1], ebuf1, esem1)
            pltpu.async_copy(vals_hbm.at[s, 2 * q + 1], vbuf1, esem1)
            pltpu.async_copy(src.at[ebuf0.at[1, 0]], gbuf1, sem1)
            pltpu.make_async_copy(src.at[ebuf0.at[0, 0]], gbuf0, sem0).wait()
            scale(gbuf0, vbuf0.at[0])
            pltpu.sync_copy(gbuf0, acc.at[ebuf0.at[0, 1]], add=True)
            pltpu.make_async_copy(edges_hbm.at[s, 0], ebuf1, esem1).wait()
            pltpu.make_async_copy(vals_hbm.at[s, 0], vbuf1, esem1).wait()
            pltpu.async_copy(src.at[ebuf1.at[0, 0]], gbuf0, sem0)
            pltpu.make_async_copy(src.at[ebuf0.at[1, 0]], gbuf1, sem1).wait()
            scale(gbuf1, vbuf0.at[1])
            pltpu.sync_copy(gbuf1, acc.at[ebuf0.at[1, 1]], add=True)
            pltpu.async_copy(edges_hbm.at[s, pn], ebuf0, esem0)
            pltpu.async_copy(vals_hbm.at[s, pn], vbuf0, esem0)
            pltpu.async_copy(src.at[ebuf1.at[1, 0]], gbuf1, sem1)
            pltpu.make_async_copy(src.at[ebuf1.at[0, 0]], gbuf0, sem0).wait()
            scale(gbuf0, vbuf1.at[0])
            pltpu.sync_copy(gbuf0, acc.at[ebuf1.at[0, 1]], add=True)
            pltpu.make_async_copy(edges_hbm.at[s, 0], ebuf0, esem0).wait()
            pltpu.make_async_copy(vals_hbm.at[s, 0], vbuf0, esem0).wait()
            pltpu.async_copy(src.at[ebuf0.at[0, 0]], gbuf0, sem0)
            pltpu.make_async_copy(src.at[ebuf1.at[1, 0]], gbuf1, sem1).wait()
            scale(gbuf1, vbuf1.at[1])
            pltpu.sync_copy(gbuf1, acc.at[ebuf1.at[1, 1]], add=True)
            return carry

        lax.fori_loop(0, NPAIR // 2, quad, 0)
        pltpu.make_async_copy(src.at[ebuf0.at[0, 0]], gbuf0, sem0).wait()
        plsc.subcore_barrier()

    gb0 = gbuf0.at[pl.ds(0, RB)]
    gb1 = gbuf1.at[pl.ds(0, RB)]

    def dump_and_clear(dst_hbm):
        for b in range(NB):
            r = row0 + b * RB
            pltpu.sync_copy(acc.at[pl.ds(r, RB)], gb0)
            pltpu.sync_copy(gb0, dst_hbm.at[c, pl.ds(r, RB)])
        zero_stripe()
        plsc.subcore_barrier()

    edge_pass(ego_hbm.at[c])
    dump_and_clear(h1_hbm)
    edge_pass(h1_hbm.at[c])
    dump_and_clear(h2_hbm)
    edge_pass(h2_hbm.at[c])

    def crow(i, carry):
        for q in range(DH // 16):
            sl = pl.ds(q * 16, 16)
            gbuf0[i, sl] = (gbuf0[i, sl] + gbuf1[i, sl] + zbuf[i, sl]) * (1.0 / 3.0)
        return carry

    for b in range(NB):
        r = row0 + b * RB
        pltpu.sync_copy(h1_hbm.at[c, pl.ds(r, RB)], gb0)
        pltpu.sync_copy(h2_hbm.at[c, pl.ds(r, RB)], gb1)
        pltpu.sync_copy(acc.at[pl.ds(r, RB)], zbuf)
        lax.fori_loop(0, RB, crow, 0)
        pltpu.sync_copy(gb0, out_hbm.at[c, pl.ds(r, RB)])


def kernel(user_embed, item_embed, adj_indices, adj_values):
    ego = jnp.concatenate([user_embed, item_embed], axis=0)
    ego_split = ego.reshape(N, NC, DH).transpose(1, 0, 2)
    ego_pad = jnp.concatenate(
        [ego_split, jnp.zeros((NC, NP - N, DH), jnp.float32)], axis=1
    )
    row = adj_indices[0].astype(jnp.int32)
    col = adj_indices[1].astype(jnp.int32)
    val = adj_values.astype(jnp.float32)
    pad = NS * EPT - E
    padrows = (N + (jnp.arange(pad, dtype=jnp.int32) % (NP - N))).astype(jnp.int32)
    colp = jnp.concatenate([col, padrows]).reshape(NS, NCH, CHUNK)
    rowp = jnp.concatenate([row, padrows]).reshape(NS, NCH, CHUNK)
    valp = jnp.concatenate([val, jnp.zeros((pad,), jnp.float32)]).reshape(NS, NCH, CHUNK)
    edges = (jnp.stack([colp, rowp], axis=2)
             .reshape(NS, NPAIR, 2, 2, CHUNK))
    valsp = valp.reshape(NS, NPAIR, 2, CHUNK)
    out, _h1, _h2 = _sc_propagate(ego_pad, edges, valsp)
    mean_emb = out[:, :N, :].transpose(1, 0, 2).reshape(N, D)
    return mean_emb[:N_USERS], mean_emb[N_USERS:]
